# Initial kernel scaffold; baseline (speedup 1.0000x reference)
import functools

import jax
import jax.numpy as jnp
from jax import lax
from jax.experimental import pallas as pl
from jax.experimental.pallas import tpu as pltpu
from jax.experimental.pallas import tpu_sc as plsc

_NUM_CORES = 2
_NUM_SUBCORES = 16
_NUM_WORKERS = _NUM_CORES * _NUM_SUBCORES
_CHUNK = 1024


def kernel(x, table):
    batch, hist = x.shape
    _, d_model = table.shape
    n = batch * hist
    idx = x.reshape(n).astype(jnp.int32)

    rows_per_worker = n // _NUM_WORKERS
    n_chunks = rows_per_worker // _CHUNK
    mesh = plsc.VectorSubcoreMesh(core_axis_name="c", subcore_axis_name="s")

    @functools.partial(
        pl.kernel,
        mesh=mesh,
        out_type=jax.ShapeDtypeStruct((n, d_model), table.dtype),
        scratch_types=[
            pltpu.VMEM((_CHUNK,), jnp.int32),
            pltpu.VMEM((_CHUNK, d_model), jnp.float32),
            pltpu.SemaphoreType.DMA,
        ],
    )
    def gather_k(table_hbm, idx_hbm, out_hbm, idx_v, rows_v, sem):
        wid = lax.axis_index("s") * _NUM_CORES + lax.axis_index("c")
        base = wid * rows_per_worker

        @pl.loop(0, n_chunks)
        def _(j):
            off = base + j * _CHUNK
            pltpu.sync_copy(idx_hbm.at[pl.ds(off, _CHUNK)], idx_v)
            pltpu.async_copy(table_hbm.at[idx_v], rows_v, sem).wait()
            pltpu.sync_copy(rows_v, out_hbm.at[pl.ds(off, _CHUNK)])

    out = gather_k(table, idx)
    return out.reshape(batch, hist, d_model)


# speedup vs baseline: 1.0484x; 1.0484x over previous
import functools

import jax
import jax.numpy as jnp
from jax import lax
from jax.experimental import pallas as pl
from jax.experimental.pallas import tpu as pltpu
from jax.experimental.pallas import tpu_sc as plsc

_NUM_CORES = 2
_NUM_SUBCORES = 16
_NUM_WORKERS = _NUM_CORES * _NUM_SUBCORES
_CHUNK = 512
_LANES = 128
_SEL_ROWS = 2048


def _select_body(pairs_ref, par_ref, out_ref):
    d = out_ref.shape[-1]
    par = par_ref[:]
    left = pairs_ref[:, :d]
    right = pairs_ref[:, d:]
    out_ref[...] = jnp.where(par == 1, right, left)


def kernel(x, table):
    batch, hist = x.shape
    vocab, d_model = table.shape
    n = batch * hist
    idx = x.reshape(n).astype(jnp.int32)
    idx2 = idx >> 1
    par = (idx & 1).reshape(n, 1)
    table2 = table.reshape(vocab // 2, 2 * d_model)

    rows_per_worker = n // _NUM_WORKERS
    n_chunks = rows_per_worker // _CHUNK
    mesh = plsc.VectorSubcoreMesh(core_axis_name="c", subcore_axis_name="s")

    @functools.partial(
        pl.kernel,
        mesh=mesh,
        out_type=jax.ShapeDtypeStruct((n, _LANES), table.dtype),
        scratch_types=[
            pltpu.VMEM((_CHUNK,), jnp.int32),
            pltpu.VMEM((_CHUNK, _LANES), jnp.float32),
            pltpu.SemaphoreType.DMA,
        ],
    )
    def gather_k(table_hbm, idx_hbm, out_hbm, idx_v, rows_v, sem):
        wid = lax.axis_index("s") * _NUM_CORES + lax.axis_index("c")
        base = wid * rows_per_worker

        @pl.loop(0, n_chunks)
        def _(j):
            off = base + j * _CHUNK
            pltpu.sync_copy(idx_hbm.at[pl.ds(off, _CHUNK)], idx_v)
            pltpu.async_copy(table_hbm.at[idx_v], rows_v, sem).wait()
            pltpu.sync_copy(rows_v, out_hbm.at[pl.ds(off, _CHUNK)])

    pairs = gather_k(table2, idx2)

    out = pl.pallas_call(
        _select_body,
        grid=(n // _SEL_ROWS,),
        in_specs=[
            pl.BlockSpec((_SEL_ROWS, _LANES), lambda i: (i, 0)),
            pl.BlockSpec((_SEL_ROWS, 1), lambda i: (i, 0)),
        ],
        out_specs=pl.BlockSpec((_SEL_ROWS, d_model), lambda i: (i, 0)),
        out_shape=jax.ShapeDtypeStruct((n, d_model), table.dtype),
    )(pairs, par)

    return out.reshape(batch, hist, d_model)


# speedup vs baseline: 1.2326x; 1.1758x over previous
import functools

import jax
import jax.numpy as jnp
from jax import lax
from jax.experimental import pallas as pl
from jax.experimental.pallas import tpu as pltpu
from jax.experimental.pallas import tpu_sc as plsc

_NUM_CORES = 2
_NUM_SUBCORES = 16
_NUM_WORKERS = _NUM_CORES * _NUM_SUBCORES
_CHUNK = 400
_LANES = 128
_SEL_BATCH = 64


def _select_body(pairs_ref, par_ref, out_ref):
    b, hist, d = out_ref.shape
    par = par_ref[:]
    left = pairs_ref[:, :d]
    right = pairs_ref[:, d:]
    out_ref[...] = jnp.where(par == 1, right, left).reshape(b, hist, d)


def kernel(x, table):
    batch, hist = x.shape
    vocab, d_model = table.shape
    n = batch * hist
    idx = x.reshape(n).astype(jnp.int32)
    idx2 = idx >> 1
    par = (idx & 1).reshape(n, 1)
    table2 = table.reshape(vocab // 2, 2 * d_model)

    rows_per_worker = n // _NUM_WORKERS
    n_chunks = rows_per_worker // _CHUNK
    assert n_chunks % 2 == 0 and n_chunks * _CHUNK == rows_per_worker
    mesh = plsc.VectorSubcoreMesh(core_axis_name="c", subcore_axis_name="s")

    @functools.partial(
        pl.kernel,
        mesh=mesh,
        out_type=jax.ShapeDtypeStruct((n, _LANES), table.dtype),
        scratch_types=[
            pltpu.VMEM((_CHUNK,), jnp.int32),
            pltpu.VMEM((_CHUNK,), jnp.int32),
            pltpu.VMEM((_CHUNK, _LANES), jnp.float32),
            pltpu.VMEM((_CHUNK, _LANES), jnp.float32),
            pltpu.SemaphoreType.DMA,
            pltpu.SemaphoreType.DMA,
            pltpu.SemaphoreType.DMA,
            pltpu.SemaphoreType.DMA,
        ],
    )
    def gather_k(table_hbm, idx_hbm, out_hbm, iv_a, iv_b, rows_a, rows_b,
                 g_a, g_b, w_a, w_b):
        wid = lax.axis_index("s") * _NUM_CORES + lax.axis_index("c")
        base = wid * rows_per_worker
        last = base + (n_chunks - 1) * _CHUNK

        def load_idx(off, iv):
            pltpu.sync_copy(idx_hbm.at[pl.ds(off, _CHUNK)], iv)

        load_idx(base, iv_a)
        pltpu.async_copy(table_hbm.at[iv_a], rows_a, g_a)
        load_idx(base + _CHUNK, iv_b)
        pltpu.async_copy(table_hbm.at[iv_b], rows_b, g_b)

        @pl.loop(0, n_chunks, step=2)
        def _(j):
            off_a = base + j * _CHUNK
            off_b = off_a + _CHUNK
            pltpu.make_async_copy(table_hbm.at[iv_a], rows_a, g_a).wait()
            pltpu.async_copy(rows_a, out_hbm.at[pl.ds(off_a, _CHUNK)], w_a)
            pltpu.make_async_copy(table_hbm.at[iv_b], rows_b, g_b).wait()
            pltpu.async_copy(rows_b, out_hbm.at[pl.ds(off_b, _CHUNK)], w_b)
            off_a2 = jnp.minimum(off_a + 2 * _CHUNK, last)
            off_b2 = jnp.minimum(off_b + 2 * _CHUNK, last)
            load_idx(off_a2, iv_a)
            pltpu.make_async_copy(rows_a, out_hbm.at[pl.ds(off_a, _CHUNK)],
                                  w_a).wait()
            pltpu.async_copy(table_hbm.at[iv_a], rows_a, g_a)
            load_idx(off_b2, iv_b)
            pltpu.make_async_copy(rows_b, out_hbm.at[pl.ds(off_b, _CHUNK)],
                                  w_b).wait()
            pltpu.async_copy(table_hbm.at[iv_b], rows_b, g_b)

        pltpu.make_async_copy(table_hbm.at[iv_a], rows_a, g_a).wait()
        pltpu.make_async_copy(table_hbm.at[iv_b], rows_b, g_b).wait()

    pairs = gather_k(table2, idx2)

    rows_per_sel = _SEL_BATCH * hist
    out = pl.pallas_call(
        _select_body,
        grid=(batch // _SEL_BATCH,),
        in_specs=[
            pl.BlockSpec((rows_per_sel, _LANES), lambda i: (i, 0)),
            pl.BlockSpec((rows_per_sel, 1), lambda i: (i, 0)),
        ],
        out_specs=pl.BlockSpec((_SEL_BATCH, hist, d_model), lambda i: (i, 0, 0)),
        out_shape=jax.ShapeDtypeStruct((batch, hist, d_model), table.dtype),
    )(pairs, par)

    return out
